# baseline (speedup 1.0000x reference)
import functools

import jax
import jax.numpy as jnp
from jax import lax
from jax.experimental import pallas as pl
from jax.experimental.pallas import tpu as pltpu
from jax.experimental.pallas import tpu_sc as plsc

B = 16384
EMBED_DIM = 32
CONTENT_DIM = 64


def _make_sc_gather():
    info = plsc.get_sparse_core_info()
    nw = info.num_cores * info.num_subcores
    b_per_w = B // nw
    mesh = plsc.VectorSubcoreMesh(core_axis_name="c", subcore_axis_name="s")

    @functools.partial(
        pl.kernel,
        mesh=mesh,
        compiler_params=pltpu.CompilerParams(use_tc_tiling_on_sc=False),
        out_type=[
            jax.ShapeDtypeStruct((B, EMBED_DIM), jnp.float32),
            jax.ShapeDtypeStruct((B, EMBED_DIM), jnp.float32),
            jax.ShapeDtypeStruct((B, CONTENT_DIM), jnp.float32),
        ],
        scratch_types=[
            pltpu.VMEM((b_per_w,), jnp.int32),
            pltpu.VMEM((b_per_w,), jnp.int32),
            pltpu.VMEM((b_per_w, EMBED_DIM), jnp.float32),
            pltpu.VMEM((b_per_w, EMBED_DIM), jnp.float32),
            pltpu.VMEM((b_per_w, CONTENT_DIM), jnp.float32),
            pltpu.SemaphoreType.DMA,
            pltpu.SemaphoreType.DMA,
            pltpu.SemaphoreType.DMA,
        ],
    )
    def gather(user_ids, item_ids, user_table, item_table, content_matrix,
               user_out, item_out, content_out,
               uidx_v, iidx_v, urows_v, irows_v, crows_v, s0, s1, s2):
        wid = lax.axis_index("s") * info.num_cores + lax.axis_index("c")
        base = wid * b_per_w
        pltpu.sync_copy(user_ids.at[pl.ds(base, b_per_w)], uidx_v)
        pltpu.sync_copy(item_ids.at[pl.ds(base, b_per_w)], iidx_v)
        cu = pltpu.async_copy(user_table.at[uidx_v], urows_v, s0)
        ci = pltpu.async_copy(item_table.at[iidx_v], irows_v, s1)
        cc = pltpu.async_copy(content_matrix.at[iidx_v], crows_v, s2)
        cu.wait()
        ci.wait()
        cc.wait()
        pltpu.sync_copy(urows_v, user_out.at[pl.ds(base, b_per_w)])
        pltpu.sync_copy(irows_v, item_out.at[pl.ds(base, b_per_w)])
        pltpu.sync_copy(crows_v, content_out.at[pl.ds(base, b_per_w)])

    return gather


_sc_gather = _make_sc_gather()


def _score_body(user_ref, item_ref, crows_ref, w_ref, b_ref, out_ref):
    proj = jnp.dot(crows_ref[...], w_ref[...], preferred_element_type=jnp.float32)
    proj = jnp.maximum(proj + b_ref[...][None, :], 0.0)
    out_ref[...] = jnp.sum(user_ref[...] * (item_ref[...] + proj), axis=1)


def _tc_score(user_vec, item_vec, crows, W, b):
    grid = 8
    blk = B // grid
    return pl.pallas_call(
        _score_body,
        grid=(grid,),
        in_specs=[
            pl.BlockSpec((blk, EMBED_DIM), lambda i: (i, 0)),
            pl.BlockSpec((blk, EMBED_DIM), lambda i: (i, 0)),
            pl.BlockSpec((blk, CONTENT_DIM), lambda i: (i, 0)),
            pl.BlockSpec((CONTENT_DIM, EMBED_DIM), lambda i: (0, 0)),
            pl.BlockSpec((EMBED_DIM,), lambda i: (0,)),
        ],
        out_specs=pl.BlockSpec((blk,), lambda i: (i,)),
        out_shape=jax.ShapeDtypeStruct((B,), jnp.float32),
    )(user_vec, item_vec, crows, W, b)


def kernel(user_ids, item_ids, user_table, item_table, content_matrix, W, b):
    user_ids = user_ids.astype(jnp.int32)
    item_ids = item_ids.astype(jnp.int32)
    user_vec, item_vec, crows = _sc_gather(
        user_ids, item_ids, user_table, item_table, content_matrix)
    return _tc_score(user_vec, item_vec, crows, W, b)
